# baseline (speedup 1.0000x reference)
import jax
import jax.numpy as jnp
from jax.experimental import pallas as pl
from jax.experimental.pallas import tpu as pltpu

_N = 10000
_F = 128
_R = 200


def _gcn2_body(s1_ref, s2_ref, wt_ref, b_ref, pw_ref, adj_ref,
               h1_ref, h2_ref, ps_ref, f_scr):
    i = pl.program_id(0)

    @pl.when(i == 0)
    def _():
        wt = wt_ref[...]
        f_scr[:, :_F] = jnp.dot(s1_ref[...], wt,
                                preferred_element_type=jnp.float32)
        f_scr[:, _F:] = jnp.dot(s2_ref[...], wt,
                                preferred_element_type=jnp.float32)

    acc = jnp.dot(adj_ref[...], f_scr[...],
                  preferred_element_type=jnp.float32)
    acc = acc + b_ref[...]
    h = jnp.where(acc > 0, acc, acc * pw_ref[...])
    h1 = h[:, :_F]
    h1_ref[0] = h1
    h2_ref[0] = h[:, _F:]
    ps_ref[0] = jnp.sum(h1, axis=0, keepdims=True)


def _disc_body(ps_ref, wb_ref, bb_ref, h1_ref, h2_ref, sc1_ref, sc2_ref):
    tot = jnp.sum(ps_ref[...], axis=0, keepdims=True)
    c = jax.nn.sigmoid(tot * (1.0 / _N))
    v = jax.lax.dot_general(c, wb_ref[...], (((1,), (1,)), ((), ())),
                            preferred_element_type=jnp.float32)
    sc1_ref[...] = jnp.sum(h1_ref[0] * v, axis=1, keepdims=True) + bb_ref[...]
    sc2_ref[...] = jnp.sum(h2_ref[0] * v, axis=1, keepdims=True) + bb_ref[...]


def kernel(cc_label, seq1, seq2, adj, sparse, msk, samp_bias1, samp_bias2,
           W_fc, b_gcn, prelu_w, W_bil, b_bil):
    s1 = seq1[0]
    s2 = seq2[0]
    A = adj[0]
    wt = W_fc.T
    b2 = jnp.concatenate([b_gcn, b_gcn])[None, :]
    pw = prelu_w.reshape(1, 1)
    bb = b_bil.reshape(1, 1)

    nb = _N // _R
    h1, h2, psums = pl.pallas_call(
        _gcn2_body,
        grid=(nb,),
        in_specs=[
            pl.BlockSpec((_N, _F), lambda i: (0, 0)),
            pl.BlockSpec((_N, _F), lambda i: (0, 0)),
            pl.BlockSpec((_F, _F), lambda i: (0, 0)),
            pl.BlockSpec((1, 2 * _F), lambda i: (0, 0)),
            pl.BlockSpec((1, 1), lambda i: (0, 0)),
            pl.BlockSpec((_R, _N), lambda i: (i, 0)),
        ],
        out_specs=[
            pl.BlockSpec((1, _R, _F), lambda i: (0, i, 0)),
            pl.BlockSpec((1, _R, _F), lambda i: (0, i, 0)),
            pl.BlockSpec((1, 1, _F), lambda i: (i, 0, 0)),
        ],
        out_shape=[
            jax.ShapeDtypeStruct((1, _N, _F), jnp.float32),
            jax.ShapeDtypeStruct((1, _N, _F), jnp.float32),
            jax.ShapeDtypeStruct((nb, 1, _F), jnp.float32),
        ],
        scratch_shapes=[pltpu.VMEM((_N, 2 * _F), jnp.float32)],
    )(s1, s2, wt, b2, pw, A)

    sc1, sc2 = pl.pallas_call(
        _disc_body,
        out_shape=[
            jax.ShapeDtypeStruct((_N, 1), jnp.float32),
            jax.ShapeDtypeStruct((_N, 1), jnp.float32),
        ],
    )(psums.reshape(nb, _F), W_bil[0], bb, h1, h2)

    ret = jnp.concatenate([sc1[:, 0][None, :] + samp_bias1,
                           sc2[:, 0][None, :] + samp_bias2], axis=1)
    return (ret, h1, h2)


# speedup vs baseline: 1.0083x; 1.0083x over previous
import jax
import jax.numpy as jnp
from jax.experimental import pallas as pl
from jax.experimental.pallas import tpu as pltpu

_N = 10000
_F = 128
_R = 400


def _gcn2_body(s1_ref, s2_ref, wt_ref, b_ref, pw_ref, adj_ref,
               h1_ref, h2_ref, ps_ref, f_scr):
    i = pl.program_id(0)

    @pl.when(i == 0)
    def _():
        wt = wt_ref[...]
        f_scr[:, :_F] = jnp.dot(s1_ref[...], wt,
                                preferred_element_type=jnp.float32)
        f_scr[:, _F:] = jnp.dot(s2_ref[...], wt,
                                preferred_element_type=jnp.float32)

    acc = jnp.dot(adj_ref[...], f_scr[...],
                  preferred_element_type=jnp.float32)
    acc = acc + b_ref[...]
    h = jnp.where(acc > 0, acc, acc * pw_ref[...])
    h1 = h[:, :_F]
    h1_ref[0] = h1
    h2_ref[0] = h[:, _F:]
    ps_ref[0] = jnp.sum(h1, axis=0, keepdims=True)


def _disc_body(ps_ref, wb_ref, bb_ref, h1_ref, h2_ref, sc1_ref, sc2_ref):
    tot = jnp.sum(ps_ref[...], axis=0, keepdims=True)
    c = jax.nn.sigmoid(tot * (1.0 / _N))
    v = jax.lax.dot_general(c, wb_ref[...], (((1,), (1,)), ((), ())),
                            preferred_element_type=jnp.float32)
    sc1_ref[...] = jnp.sum(h1_ref[0] * v, axis=1, keepdims=True) + bb_ref[...]
    sc2_ref[...] = jnp.sum(h2_ref[0] * v, axis=1, keepdims=True) + bb_ref[...]


def kernel(cc_label, seq1, seq2, adj, sparse, msk, samp_bias1, samp_bias2,
           W_fc, b_gcn, prelu_w, W_bil, b_bil):
    s1 = seq1[0]
    s2 = seq2[0]
    A = adj[0]
    wt = W_fc.T
    b2 = jnp.concatenate([b_gcn, b_gcn])[None, :]
    pw = prelu_w.reshape(1, 1)
    bb = b_bil.reshape(1, 1)

    nb = _N // _R
    h1, h2, psums = pl.pallas_call(
        _gcn2_body,
        grid=(nb,),
        in_specs=[
            pl.BlockSpec((_N, _F), lambda i: (0, 0)),
            pl.BlockSpec((_N, _F), lambda i: (0, 0)),
            pl.BlockSpec((_F, _F), lambda i: (0, 0)),
            pl.BlockSpec((1, 2 * _F), lambda i: (0, 0)),
            pl.BlockSpec((1, 1), lambda i: (0, 0)),
            pl.BlockSpec((_R, _N), lambda i: (i, 0)),
        ],
        out_specs=[
            pl.BlockSpec((1, _R, _F), lambda i: (0, i, 0)),
            pl.BlockSpec((1, _R, _F), lambda i: (0, i, 0)),
            pl.BlockSpec((1, 1, _F), lambda i: (i, 0, 0)),
        ],
        out_shape=[
            jax.ShapeDtypeStruct((1, _N, _F), jnp.float32),
            jax.ShapeDtypeStruct((1, _N, _F), jnp.float32),
            jax.ShapeDtypeStruct((nb, 1, _F), jnp.float32),
        ],
        scratch_shapes=[pltpu.VMEM((_N, 2 * _F), jnp.float32)],
    )(s1, s2, wt, b2, pw, A)

    sc1, sc2 = pl.pallas_call(
        _disc_body,
        out_shape=[
            jax.ShapeDtypeStruct((_N, 1), jnp.float32),
            jax.ShapeDtypeStruct((_N, 1), jnp.float32),
        ],
    )(psums.reshape(nb, _F), W_bil[0], bb, h1, h2)

    ret = jnp.concatenate([sc1[:, 0][None, :] + samp_bias1,
                           sc2[:, 0][None, :] + samp_bias2], axis=1)
    return (ret, h1, h2)


# speedup vs baseline: 1.0310x; 1.0225x over previous
import jax
import jax.numpy as jnp
from jax.experimental import pallas as pl
from jax.experimental.pallas import tpu as pltpu

_N = 10000
_F = 128
_R = 200
_NB = _N // _R


def _dgi_body(s1_ref, s2_ref, wt_ref, b_ref, pw_ref, wb_ref, bb_ref, adj_ref,
              h1_ref, h2_ref, sc1_ref, sc2_ref, f_scr, ps_scr):
    i = pl.program_id(0)

    @pl.when(i == 0)
    def _():
        wt = wt_ref[...]
        f_scr[:, :_F] = jnp.dot(s1_ref[...], wt,
                                preferred_element_type=jnp.float32)
        f_scr[:, _F:] = jnp.dot(s2_ref[...], wt,
                                preferred_element_type=jnp.float32)
        ps_scr[...] = jnp.zeros_like(ps_scr)

    acc = jnp.dot(adj_ref[...], f_scr[...],
                  preferred_element_type=jnp.float32)
    acc = acc + b_ref[...]
    h = jnp.where(acc > 0, acc, acc * pw_ref[...])
    h1 = h[:, :_F]
    h1_ref[0, pl.ds(i * _R, _R), :] = h1
    h2_ref[0, pl.ds(i * _R, _R), :] = h[:, _F:]
    ps_scr[...] += jnp.sum(h1, axis=0, keepdims=True)

    @pl.when(i == _NB - 1)
    def _():
        c = jax.nn.sigmoid(ps_scr[...] * (1.0 / _N))
        v = jax.lax.dot_general(c, wb_ref[...], (((1,), (1,)), ((), ())),
                                preferred_element_type=jnp.float32)
        sc1_ref[...] = jnp.sum(h1_ref[0] * v, axis=1, keepdims=True) + bb_ref[...]
        sc2_ref[...] = jnp.sum(h2_ref[0] * v, axis=1, keepdims=True) + bb_ref[...]


def kernel(cc_label, seq1, seq2, adj, sparse, msk, samp_bias1, samp_bias2,
           W_fc, b_gcn, prelu_w, W_bil, b_bil):
    s1 = seq1[0]
    s2 = seq2[0]
    A = adj[0]
    wt = W_fc.T
    b2 = jnp.concatenate([b_gcn, b_gcn])[None, :]
    pw = prelu_w.reshape(1, 1)
    bb = b_bil.reshape(1, 1)

    h1, h2, sc1, sc2 = pl.pallas_call(
        _dgi_body,
        grid=(_NB,),
        in_specs=[
            pl.BlockSpec((_N, _F), lambda i: (0, 0)),
            pl.BlockSpec((_N, _F), lambda i: (0, 0)),
            pl.BlockSpec((_F, _F), lambda i: (0, 0)),
            pl.BlockSpec((1, 2 * _F), lambda i: (0, 0)),
            pl.BlockSpec((1, 1), lambda i: (0, 0)),
            pl.BlockSpec((_F, _F), lambda i: (0, 0)),
            pl.BlockSpec((1, 1), lambda i: (0, 0)),
            pl.BlockSpec((_R, _N), lambda i: (i, 0)),
        ],
        out_specs=[
            pl.BlockSpec((1, _N, _F), lambda i: (0, 0, 0)),
            pl.BlockSpec((1, _N, _F), lambda i: (0, 0, 0)),
            pl.BlockSpec((_N, 1), lambda i: (0, 0)),
            pl.BlockSpec((_N, 1), lambda i: (0, 0)),
        ],
        out_shape=[
            jax.ShapeDtypeStruct((1, _N, _F), jnp.float32),
            jax.ShapeDtypeStruct((1, _N, _F), jnp.float32),
            jax.ShapeDtypeStruct((_N, 1), jnp.float32),
            jax.ShapeDtypeStruct((_N, 1), jnp.float32),
        ],
        scratch_shapes=[
            pltpu.VMEM((_N, 2 * _F), jnp.float32),
            pltpu.VMEM((1, _F), jnp.float32),
        ],
    )(s1, s2, wt, b2, pw, W_bil[0], bb, A)

    ret = jnp.concatenate([sc1[:, 0][None, :] + samp_bias1,
                           sc2[:, 0][None, :] + samp_bias2], axis=1)
    return (ret, h1, h2)
